# baseline (speedup 1.0000x reference)
import jax
import jax.numpy as jnp
from jax import lax
from jax.experimental import pallas as pl
from jax.experimental.pallas import tpu as pltpu
from jax.experimental.pallas import tpu_sc as plsc

B, L, D = 4096, 200, 64
NC, NS = 2, 16
NW = NC * NS
F = L * D
NBW = 8
BW = B // NBW
NFP = NW // NBW
FP = F // NFP
FC = 40
NBLK = FP // FC
NBUF = 4
NG = BW // 16


def _body(idx_hbm, emb_hbm, out_hbm, m_v, ib0, ib1, bufs, thr_s, lane_s,
          is0, is1, in_sems, out_sems):
    wid = lax.axis_index("s") * NC + lax.axis_index("c")
    bw, fp = wid // NFP, wid % NFP
    b0 = bw * BW
    f_base = fp * FP

    def in_desc(blk, s):
        return pltpu.make_async_copy(
            emb_hbm.at[pl.ds(f_base + blk * FC, FC), pl.ds(b0, BW)],
            bufs[s], in_sems[s])

    def out_desc(blk, s):
        return pltpu.make_async_copy(
            bufs[s], out_hbm.at[pl.ds(f_base + blk * FC, FC), pl.ds(b0, BW)],
            out_sems[s])

    in_desc(0, 0).start()
    in_desc(1, 1).start()

    lvec_full = jnp.full((16,), L, jnp.int32)

    def _minit(g, _):
        m_v[pl.ds(g * 16, 16)] = lvec_full
        return 0
    lax.fori_loop(0, NG, _minit, 0)

    ibs = (ib0, ib1)
    isems = (is0, is1)

    def idx_desc(c, s):
        return pltpu.make_async_copy(
            idx_hbm.at[pl.ds(c * 8, 8), pl.ds(b0, BW)], ibs[s], isems[s])

    idx_desc(0, 0).start()

    def _chunk(c, _):
        for s in range(2):
            cc = c * 2 + s

            @pl.when(cc + 1 < L // 8)
            def _pre(cc=cc, s=s):
                idx_desc(cc + 1, 1 - s).start()
            idx_desc(cc, s).wait()

            def _g(g, _, s=s, cc=cc):
                m = m_v[pl.ds(g * 16, 16)]
                for l_loc in range(8):
                    v = ibs[s][l_loc, pl.ds(g * 16, 16)]
                    m = jnp.minimum(
                        m, jnp.where(v == 0, cc * 8 + l_loc, L))
                m_v[pl.ds(g * 16, 16)] = m
                return 0
            lax.fori_loop(0, NG, _g, 0)
        return 0
    lax.fori_loop(0, L // 16, _chunk, 0)

    idx_desc(L // 8 - 1, 0).wait()

    def _g_last(g, _):
        m = m_v[pl.ds(g * 16, 16)]
        for l_loc in range(8):
            v = ibs[0][l_loc, pl.ds(g * 16, 16)]
            m = jnp.minimum(m, jnp.where(v == 0, (L // 8 - 1) * 8 + l_loc, L))
        m_v[pl.ds(g * 16, 16)] = m
        return 0
    lax.fori_loop(0, NG, _g_last, 0)

    def _park(g, np_):
        m = m_v[pl.ds(g * 16, 16)]
        for j in range(16):
            c = m[j]
            bl = g * 16 + j
            thr_s[bl] = c * D

            @pl.when(c < L)
            def _add(np_=np_, bl=bl):
                lane_s[np_] = bl
            np_ = jnp.where(c < L, np_ + 1, np_)
        return np_
    np_ = lax.fori_loop(0, NG, _park, jnp.int32(0))

    zero16 = jnp.zeros((16,), jnp.float32)
    iota16 = lax.iota(jnp.int32, 16)

    def _compute(blk, s):
        f0 = f_base + blk * FC

        def _lane(i, _):
            bl = lane_s[i]
            lo = jnp.maximum(thr_s[bl] - f0, 0)

            @pl.when(lo < FC)
            def _zero(bl=bl, lo=lo):
                bvec = jnp.full((16,), 0, jnp.int32) + bl

                def _zchunk(k, _):
                    fvals = lo + k * 16 + iota16
                    plsc.store_scatter(bufs[s], [fvals, bvec], zero16,
                                       mask=fvals < FC)
                    return 0
                lax.fori_loop(0, (FC - lo + 15) >> 4, _zchunk, 0)
            return 0
        lax.fori_loop(0, np_, _lane, 0)

    def _group(g, _):
        for s in range(NBUF):
            blk = g * NBUF + s
            nxt = (s + 2) % NBUF

            @pl.when(blk + 2 < NBLK)
            def _pre(blk=blk, nxt=nxt):
                @pl.when(blk + 2 >= NBUF)
                def _drain(blk=blk, nxt=nxt):
                    out_desc(blk + 2 - NBUF, nxt).wait()
                in_desc(blk + 2, nxt).start()

            in_desc(blk, s).wait()
            _compute(blk, s)
            out_desc(blk, s).start()
        return 0

    lax.fori_loop(0, NBLK // NBUF, _group, 0)

    out_desc(NBLK - 2, (NBLK - 2) % NBUF).wait()
    out_desc(NBLK - 1, (NBLK - 1) % NBUF).wait()


@jax.jit
def _run(idx_t, emb_t):
    mesh = plsc.VectorSubcoreMesh(core_axis_name="c", subcore_axis_name="s",
                                  num_cores=NC, num_subcores=NS)

    def body(idx_hbm, emb_hbm, out_hbm, m_v, ib0, ib1,
             b0_, b1_, b2_, b3_, thr_s, lane_s,
             is0, is1, i0, i1, i2, i3, o0, o1, o2, o3):
        _body(idx_hbm, emb_hbm, out_hbm, m_v, ib0, ib1,
              (b0_, b1_, b2_, b3_), thr_s, lane_s,
              is0, is1, (i0, i1, i2, i3), (o0, o1, o2, o3))

    return pl.kernel(
        body,
        out_type=jax.ShapeDtypeStruct((F, B), jnp.float32),
        mesh=mesh,
        scratch_types=(
            [pltpu.VMEM((BW,), jnp.int32)]
            + [pltpu.VMEM((8, BW), jnp.int32)] * 2
            + [pltpu.VMEM((FC, BW), jnp.float32)] * NBUF
            + [pltpu.SMEM((BW,), jnp.int32), pltpu.SMEM((BW,), jnp.int32)]
            + [pltpu.SemaphoreType.DMA] * (2 + 2 * NBUF)
        ),
        compiler_params=pltpu.CompilerParams(needs_layout_passes=False),
    )(idx_t, emb_t)


def kernel(indexes, embeds):
    idx_t = indexes.astype(jnp.int32).transpose(1, 0)
    emb_t = embeds.transpose(1, 2, 0).reshape(F, B)
    out_t = _run(idx_t, emb_t)
    return out_t.reshape(L, D, B).transpose(2, 0, 1)


# speedup vs baseline: 1.5554x; 1.5554x over previous
import jax
import jax.numpy as jnp
from jax import lax
from jax.experimental import pallas as pl
from jax.experimental.pallas import tpu as pltpu
from jax.experimental.pallas import tpu_sc as plsc

B, L, D = 4096, 200, 64
NC, NS = 2, 16
NW = NC * NS
BW = B // NW
F = L * D
FC = 160
NBLK = F // FC
NBUF = 5


def _body(idx_hbm, emb_hbm, out_hbm, idx_v, bufs, thr_s, lane_s, in_sems, out_sems):
    wid = lax.axis_index("s") * NC + lax.axis_index("c")
    b0 = wid * BW

    def in_desc(blk, s):
        return pltpu.make_async_copy(
            emb_hbm.at[pl.ds(blk * FC, FC), pl.ds(b0, BW)],
            bufs[s], in_sems[s])

    def out_desc(blk, s):
        return pltpu.make_async_copy(
            bufs[s], out_hbm.at[pl.ds(blk * FC, FC), pl.ds(b0, BW)],
            out_sems[s])

    in_desc(0, 0).start()
    in_desc(1, 1).start()
    in_desc(2, 2).start()

    pltpu.sync_copy(idx_hbm.at[:, pl.ds(b0, BW)], idx_v)

    def _scan_l(l, ms):
        lvec = jnp.full((16,), 0, jnp.int32) + l
        return tuple(
            jnp.minimum(ms[g],
                        jnp.where(idx_v[l, pl.ds(g * 16, 16)] == 0, lvec, L))
            for g in range(8))
    ms = lax.fori_loop(0, L, _scan_l,
                       tuple(jnp.full((16,), L, jnp.int32) for _ in range(8)))

    np_ = jnp.int32(0)
    for g in range(8):
        for j in range(16):
            c = ms[g][j]
            bl = g * 16 + j
            thr_s[bl] = c * D

            @pl.when(c < L)
            def _add(np_=np_, bl=bl):
                lane_s[np_] = bl
            np_ = jnp.where(c < L, np_ + 1, np_)

    zero16 = jnp.zeros((16,), jnp.float32)
    iota16 = lax.iota(jnp.int32, 16)

    def _compute(blk, s):
        f0 = blk * FC

        def _lane(i, _):
            bl = lane_s[i]
            lo = jnp.maximum(thr_s[bl] - f0, 0)

            @pl.when(lo < FC)
            def _zero(bl=bl, lo=lo):
                bvec = jnp.full((16,), 0, jnp.int32) + bl

                def _chunk(k, _):
                    fvals = lo + k * 16 + iota16
                    plsc.store_scatter(bufs[s], [fvals, bvec], zero16,
                                       mask=fvals < FC)
                    return 0
                lax.fori_loop(0, (FC - lo + 15) >> 4, _chunk, 0)
            return 0
        lax.fori_loop(0, np_, _lane, 0)

    def _group(g, _):
        for s in range(NBUF):
            blk = g * NBUF + s
            nxt = (s + 3) % NBUF

            @pl.when(blk + 3 < NBLK)
            def _pre(blk=blk, nxt=nxt):
                @pl.when(blk + 3 >= NBUF)
                def _drain(blk=blk, nxt=nxt):
                    out_desc(blk + 3 - NBUF, nxt).wait()
                in_desc(blk + 3, nxt).start()

            in_desc(blk, s).wait()
            _compute(blk, s)
            out_desc(blk, s).start()
        return 0

    lax.fori_loop(0, NBLK // NBUF, _group, 0)

    for j in range(NBLK - NBUF, NBLK):
        out_desc(j, j % NBUF).wait()


@jax.jit
def _run(idx_t, emb_t):
    mesh = plsc.VectorSubcoreMesh(core_axis_name="c", subcore_axis_name="s",
                                  num_cores=NC, num_subcores=NS)

    def body(idx_hbm, emb_hbm, out_hbm, idx_v,
             b0_, b1_, b2_, b3_, b4_, thr_s, lane_s,
             i0, i1, i2, i3, i4, o0, o1, o2, o3, o4):
        _body(idx_hbm, emb_hbm, out_hbm, idx_v,
              (b0_, b1_, b2_, b3_, b4_), thr_s, lane_s,
              (i0, i1, i2, i3, i4), (o0, o1, o2, o3, o4))

    return pl.kernel(
        body,
        out_type=jax.ShapeDtypeStruct((F, B), jnp.float32),
        mesh=mesh,
        scratch_types=(
            [pltpu.VMEM((L, BW), jnp.int32)]
            + [pltpu.VMEM((FC, BW), jnp.float32)] * NBUF
            + [pltpu.SMEM((BW,), jnp.int32), pltpu.SMEM((BW,), jnp.int32)]
            + [pltpu.SemaphoreType.DMA] * (2 * NBUF)
        ),
        compiler_params=pltpu.CompilerParams(needs_layout_passes=False),
    )(idx_t, emb_t)


def kernel(indexes, embeds):
    idx_t = indexes.astype(jnp.int32).transpose(1, 0)
    emb_t = embeds.transpose(1, 2, 0).reshape(F, B)
    out_t = _run(idx_t, emb_t)
    return out_t.reshape(L, D, B).transpose(2, 0, 1)
